# Initial kernel scaffold; baseline (speedup 1.0000x reference)
import jax
import jax.numpy as jnp
from jax.experimental import pallas as pl


def kernel(word, pos1, pos2, chars, word_table, pos1_table, pos2_table):
    raise NotImplementedError("write your pallas kernel here")


# speedup vs baseline: 3.6204x; 3.6204x over previous
import functools

import jax
import jax.numpy as jnp
from jax import lax
from jax.experimental import pallas as pl
from jax.experimental.pallas import tpu as pltpu
from jax.experimental.pallas import tpu_sc as plsc

B, S = 4096, 200
WORD_SIZE, POS_SIZE = 64, 16
OUT_SIZE = WORD_SIZE + 2 * POS_SIZE
N = B * S

NC, NS = 2, 16
NW = NC * NS
TOK_PER_W = N // NW

IDX_W = 128
CHUNK = 512
SUB = CHUNK // IDX_W
NCHUNK = TOK_PER_W // CHUNK


def _emb_body(word_hbm, pos1_hbm, pos2_hbm, wt_hbm, p1t_hbm, p2t_hbm,
              out_hbm, idxw_v, idx1_v, idx2_v, w_v, p1_v, p2_v, sem):
    wid = lax.axis_index("s") * NC + lax.axis_index("c")
    row0 = wid * (TOK_PER_W // IDX_W)

    @pl.loop(0, NCHUNK)
    def _(i):
        irow = row0 + i * SUB
        base = irow * IDX_W
        pltpu.sync_copy(word_hbm.at[pl.ds(irow, SUB)], idxw_v)
        pltpu.sync_copy(pos1_hbm.at[pl.ds(irow, SUB)], idx1_v)
        pltpu.sync_copy(pos2_hbm.at[pl.ds(irow, SUB)], idx2_v)
        for j in range(SUB):
            dst = pl.ds(j * IDX_W, IDX_W)
            cw = pltpu.async_copy(wt_hbm.at[idxw_v.at[j]], w_v.at[dst], sem)
            c1 = pltpu.async_copy(p1t_hbm.at[idx1_v.at[j]], p1_v.at[dst], sem)
            c2 = pltpu.async_copy(p2t_hbm.at[idx2_v.at[j]], p2_v.at[dst], sem)
            cw.wait()
            c1.wait()
            c2.wait()
        pltpu.sync_copy(w_v, out_hbm.at[pl.ds(base, CHUNK), pl.ds(0, WORD_SIZE)])
        pltpu.sync_copy(p1_v, out_hbm.at[pl.ds(base, CHUNK),
                                         pl.ds(WORD_SIZE, POS_SIZE)])
        pltpu.sync_copy(p2_v, out_hbm.at[pl.ds(base, CHUNK),
                                         pl.ds(WORD_SIZE + POS_SIZE, POS_SIZE)])


@jax.jit
def _embed(word, pos1, pos2, word_table, pos1_table, pos2_table):
    mesh = plsc.VectorSubcoreMesh(core_axis_name="c", subcore_axis_name="s")
    k = pl.kernel(
        _emb_body,
        out_type=jax.ShapeDtypeStruct((N, OUT_SIZE), jnp.float32),
        mesh=mesh,
        compiler_params=pltpu.CompilerParams(use_tc_tiling_on_sc=False),
        scratch_types=[
            pltpu.VMEM((SUB, IDX_W), jnp.int32),
            pltpu.VMEM((SUB, IDX_W), jnp.int32),
            pltpu.VMEM((SUB, IDX_W), jnp.int32),
            pltpu.VMEM((CHUNK, WORD_SIZE), jnp.float32),
            pltpu.VMEM((CHUNK, POS_SIZE), jnp.float32),
            pltpu.VMEM((CHUNK, POS_SIZE), jnp.float32),
            pltpu.SemaphoreType.DMA,
        ],
    )
    wi = word.reshape(N // IDX_W, IDX_W)
    p1i = pos1.reshape(N // IDX_W, IDX_W)
    p2i = pos2.reshape(N // IDX_W, IDX_W)
    out = k(wi, p1i, p2i, word_table, pos1_table, pos2_table)
    return out.reshape(B, S, OUT_SIZE)


def kernel(word, pos1, pos2, chars, word_table, pos1_table, pos2_table):
    del chars
    return _embed(word, pos1, pos2, word_table, pos1_table, pos2_table)


# speedup vs baseline: 4.0747x; 1.1255x over previous
import jax
import jax.numpy as jnp
from jax import lax
from jax.experimental import pallas as pl
from jax.experimental.pallas import tpu as pltpu
from jax.experimental.pallas import tpu_sc as plsc

B, S = 4096, 200
WORD_SIZE, POS_SIZE = 64, 16
OUT_SIZE = WORD_SIZE + 2 * POS_SIZE
N = B * S

NC, NS = 2, 16
NW = NC * NS
TOK_PER_W = N // NW

IDX_W = 128
CHUNK = 512
SUB = CHUNK // IDX_W
NCHUNK = TOK_PER_W // CHUNK
ROWS_PER_W = TOK_PER_W // IDX_W


def _emb_body(word_hbm, pos1_hbm, pos2_hbm, wt_hbm, p1t_hbm, p2t_hbm,
              out_hbm, idxw_v, idx1_v, idx2_v, w_v, p1_v, p2_v,
              semL0, semL1, semG0, semG1, semW0, semW1):
    semL = (semL0, semL1)
    semG = (semG0, semG1)
    semW = (semW0, semW1)
    wid = lax.axis_index("s") * NC + lax.axis_index("c")
    row0 = wid * ROWS_PER_W

    def fire_L(i, b):
        irow = row0 + i * SUB
        pltpu.async_copy(word_hbm.at[pl.ds(irow, SUB)], idxw_v.at[b], semL[b])
        pltpu.async_copy(pos1_hbm.at[pl.ds(irow, SUB)], idx1_v.at[b], semL[b])
        pltpu.async_copy(pos2_hbm.at[pl.ds(irow, SUB)], idx2_v.at[b], semL[b])

    def wait_L(b):
        pltpu.make_async_copy(word_hbm.at[pl.ds(0, SUB)], idxw_v.at[b],
                              semL[b]).wait()
        pltpu.make_async_copy(pos1_hbm.at[pl.ds(0, SUB)], idx1_v.at[b],
                              semL[b]).wait()
        pltpu.make_async_copy(pos2_hbm.at[pl.ds(0, SUB)], idx2_v.at[b],
                              semL[b]).wait()

    def fire_G(b):
        for j in range(SUB):
            dst = pl.ds(j * IDX_W, IDX_W)
            pltpu.async_copy(wt_hbm.at[idxw_v.at[b].at[j]],
                             w_v.at[b].at[dst], semG[b])
            pltpu.async_copy(p1t_hbm.at[idx1_v.at[b].at[j]],
                             p1_v.at[b].at[dst], semG[b])
            pltpu.async_copy(p2t_hbm.at[idx2_v.at[b].at[j]],
                             p2_v.at[b].at[dst], semG[b])

    def wait_G(b):
        for j in range(SUB):
            dst = pl.ds(j * IDX_W, IDX_W)
            pltpu.make_async_copy(wt_hbm.at[idxw_v.at[b].at[j]],
                                  w_v.at[b].at[dst], semG[b]).wait()
            pltpu.make_async_copy(p1t_hbm.at[idx1_v.at[b].at[j]],
                                  p1_v.at[b].at[dst], semG[b]).wait()
            pltpu.make_async_copy(p2t_hbm.at[idx2_v.at[b].at[j]],
                                  p2_v.at[b].at[dst], semG[b]).wait()

    def out_slices(i):
        base = (row0 + i * SUB) * IDX_W
        rows = pl.ds(base, CHUNK)
        return (out_hbm.at[rows, pl.ds(0, WORD_SIZE)],
                out_hbm.at[rows, pl.ds(WORD_SIZE, POS_SIZE)],
                out_hbm.at[rows, pl.ds(WORD_SIZE + POS_SIZE, POS_SIZE)])

    def fire_W(i, b):
        ow, o1, o2 = out_slices(i)
        pltpu.async_copy(w_v.at[b], ow, semW[b])
        pltpu.async_copy(p1_v.at[b], o1, semW[b])
        pltpu.async_copy(p2_v.at[b], o2, semW[b])

    def wait_W(b):
        ow, o1, o2 = out_slices(0)
        pltpu.make_async_copy(w_v.at[b], ow, semW[b]).wait()
        pltpu.make_async_copy(p1_v.at[b], o1, semW[b]).wait()
        pltpu.make_async_copy(p2_v.at[b], o2, semW[b]).wait()

    fire_L(0, 0)
    wait_L(0)
    fire_G(0)
    fire_L(1, 1)

    wait_G(0)
    fire_W(0, 0)
    wait_L(1)
    fire_G(1)
    fire_L(2, 0)

    def step(i, b):
        wait_G(b)
        fire_W(i, b)
        wait_L(1 - b)
        wait_W(1 - b)
        fire_G(1 - b)
        fire_L(jnp.minimum(i + 2, NCHUNK - 1), b)

    @pl.loop(0, (NCHUNK - 2) // 2)
    def _(k):
        step(2 * k + 1, 1)
        step(2 * k + 2, 0)

    wait_G(1)
    fire_W(NCHUNK - 1, 1)
    wait_W(0)
    wait_W(1)
    wait_L(0)


@jax.jit
def _embed(word, pos1, pos2, word_table, pos1_table, pos2_table):
    mesh = plsc.VectorSubcoreMesh(core_axis_name="c", subcore_axis_name="s")
    k = pl.kernel(
        _emb_body,
        out_type=jax.ShapeDtypeStruct((N, OUT_SIZE), jnp.float32),
        mesh=mesh,
        compiler_params=pltpu.CompilerParams(use_tc_tiling_on_sc=False),
        scratch_types=[
            pltpu.VMEM((2, SUB, IDX_W), jnp.int32),
            pltpu.VMEM((2, SUB, IDX_W), jnp.int32),
            pltpu.VMEM((2, SUB, IDX_W), jnp.int32),
            pltpu.VMEM((2, CHUNK, WORD_SIZE), jnp.float32),
            pltpu.VMEM((2, CHUNK, POS_SIZE), jnp.float32),
            pltpu.VMEM((2, CHUNK, POS_SIZE), jnp.float32),
            pltpu.SemaphoreType.DMA,
            pltpu.SemaphoreType.DMA,
            pltpu.SemaphoreType.DMA,
            pltpu.SemaphoreType.DMA,
            pltpu.SemaphoreType.DMA,
            pltpu.SemaphoreType.DMA,
        ],
    )
    wi = word.reshape(N // IDX_W, IDX_W)
    p1i = pos1.reshape(N // IDX_W, IDX_W)
    p2i = pos2.reshape(N // IDX_W, IDX_W)
    out = k(wi, p1i, p2i, word_table, pos1_table, pos2_table)
    return out.reshape(B, S, OUT_SIZE)


def kernel(word, pos1, pos2, chars, word_table, pos1_table, pos2_table):
    del chars
    return _embed(word, pos1, pos2, word_table, pos1_table, pos2_table)
